# baseline (speedup 1.0000x reference)
import jax
import jax.numpy as jnp
from jax.experimental import pallas as pl
from jax.experimental.pallas import tpu as pltpu


def _compiler_params(dims):
    return pltpu.CompilerParams(
        dimension_semantics=dims,
        vmem_limit_bytes=55 * 1024 * 1024,
    )


def _conv_gather_kernel(lin_ref, x_ref, w1_ref, b1_ref, w2_ref,
                        g_ref, feat_ref, xp_ref):
    b = pl.program_id(0)
    H = x_ref.shape[1]
    W = x_ref.shape[2]
    C = x_ref.shape[3]
    D = w2_ref.shape[1]
    M = g_ref.shape[1]

    @pl.when(b == 0)
    def _():
        Wp = W + 16
        xp_ref[0:1, :, :] = jnp.zeros((1, Wp, C), jnp.float32)
        xp_ref[H + 1:H + 2, :, :] = jnp.zeros((1, Wp, C), jnp.float32)
        xp_ref[:, 0:8, :] = jnp.zeros((H + 2, 8, C), jnp.float32)
        xp_ref[:, W + 8:Wp, :] = jnp.zeros((H + 2, 8, C), jnp.float32)

    xp_ref[1:H + 1, 8:W + 8, :] = x_ref[0]

    Hh = H // 2
    for half in range(2):
        acc = None
        for dy in range(3):
            for dx in range(3):
                r0 = dy + half * Hh
                xs = xp_ref[r0:r0 + Hh, dx + 7:dx + 7 + W, :]
                d = jnp.dot(xs.reshape(Hh * W, C), w1_ref[dy * 3 + dx],
                            preferred_element_type=jnp.float32)
                acc = d if acc is None else acc + d
        f1 = jnp.maximum(acc + b1_ref[...], 0.0)
        feat_ref[half * Hh * W:(half + 1) * Hh * W, :] = jnp.dot(
            f1, w2_ref[...], preferred_element_type=jnp.float32)

    iota8 = jax.lax.broadcasted_iota(jnp.int32, (8, D), 0)

    def outer(o, carry):
        m0 = pl.multiple_of(o * 16, 8)
        rows = []
        for i in range(16):
            r = lin_ref[b, m0 + i]
            base = pl.multiple_of((r >> 3) << 3, 8)
            chunk = feat_ref[pl.ds(base, 8), :]
            msk = (iota8 == (r & 7)).astype(jnp.float32)
            rows.append(jnp.sum(chunk * msk, axis=0, keepdims=True))
        g_ref[0, pl.ds(m0, 16), :] = jnp.concatenate(rows, axis=0)
        return carry

    jax.lax.fori_loop(0, M // 16, outer, 0)


def _conv_gather(x_t, lin, w1r, b1r, w2t, *, interpret=False):
    B, H, W, C = x_t.shape
    M = lin.shape[1]
    D = w2t.shape[1]
    f32 = jnp.float32
    return pl.pallas_call(
        _conv_gather_kernel,
        grid=(B,),
        in_specs=[
            pl.BlockSpec(memory_space=pltpu.SMEM),
            pl.BlockSpec((1, H, W, C), lambda b: (b, 0, 0, 0)),
            pl.BlockSpec((9, C, C), lambda b: (0, 0, 0)),
            pl.BlockSpec((1, C), lambda b: (0, 0)),
            pl.BlockSpec((C, D), lambda b: (0, 0)),
        ],
        out_specs=pl.BlockSpec((1, M, D), lambda b: (b, 0, 0)),
        out_shape=jax.ShapeDtypeStruct((B, M, D), f32),
        scratch_shapes=[pltpu.VMEM((H * W, D), f32),
                        pltpu.VMEM((H + 2, W + 16, C), f32)],
        compiler_params=_compiler_params(("parallel",)),
        interpret=interpret,
    )(lin, x_t, w1r, b1r, w2t)


def _lstm_kernel(seq_ref, idx_ref, b2_ref, ewih_ref, ewhh_ref, eb_ref,
                 dwih_ref, dwhh_ref, db_ref, pw1_ref, pb1_ref, pw2_ref,
                 pb2_ref, y_ref, gt_ref, enc_ref):
    T = seq_ref.shape[0]
    Sh = seq_ref.shape[1]
    HID = ewhh_ref.shape[0]

    gt_ref[...] = idx_ref[...].astype(jnp.float32) * 4.0

    def sig(v):
        return jnp.tanh(v * 0.5) * 0.5 + 0.5

    def gates(xt, h, c, wih, whh, bias):
        g = (jnp.dot(xt, wih, preferred_element_type=jnp.float32)
             + jnp.dot(h, whh, preferred_element_type=jnp.float32)
             + bias)
        ig = sig(g[:, 0:HID])
        fg = sig(g[:, HID:2 * HID])
        gg = jnp.tanh(g[:, 2 * HID:3 * HID])
        og = sig(g[:, 3 * HID:4 * HID])
        c = fg * c + ig * gg
        h = og * jnp.tanh(c)
        return h, c

    def enc_step(t, hc):
        h, c = gates(seq_ref[t] + b2_ref[...], hc[0], hc[1], ewih_ref[...],
                     ewhh_ref[...], eb_ref[...])
        enc_ref[t] = h
        return (h, c)

    z = jnp.zeros((Sh, HID), jnp.float32)
    hc = jax.lax.fori_loop(0, T, enc_step, (z, z))

    def dec_step(t, hc):
        h, c = gates(enc_ref[t], hc[0], hc[1], dwih_ref[...], dwhh_ref[...],
                     db_ref[...])
        m = jnp.maximum(jnp.dot(h, pw1_ref[...],
                                preferred_element_type=jnp.float32)
                        + pb1_ref[...], 0.0)
        y = jnp.dot(m, pw2_ref[...],
                    preferred_element_type=jnp.float32) + pb2_ref[0]
        t0 = pl.multiple_of(t * Sh, 8)
        y_ref[pl.ds(t0, Sh), :] = y
        return (h, c)

    jax.lax.fori_loop(0, T, dec_step, hc)


def _lstm_mlp(seq, idx_flat, b2r, ewih_t, ewhh_t, eb, dwih_t, dwhh_t, db,
              pw1t, pb1, pw2p, pb2, *, interpret=False):
    T, S, D = seq.shape
    HID = ewhh_t.shape[0]
    f32 = jnp.float32
    vmem = [pl.BlockSpec(memory_space=pltpu.VMEM)] * 12
    return pl.pallas_call(
        _lstm_kernel,
        in_specs=vmem + [pl.BlockSpec(memory_space=pltpu.SMEM)],
        out_specs=[pl.BlockSpec(memory_space=pltpu.VMEM),
                   pl.BlockSpec(memory_space=pltpu.VMEM)],
        out_shape=[jax.ShapeDtypeStruct((T * S, 128), f32),
                   jax.ShapeDtypeStruct(idx_flat.shape, f32)],
        scratch_shapes=[pltpu.VMEM((T, S, HID), f32)],
        compiler_params=_compiler_params(()),
        interpret=interpret,
    )(seq, idx_flat, b2r, ewih_t, ewhh_t, eb, dwih_t, dwhh_t, db, pw1t, pb1,
      pw2p, pb2)


def _forward(x, kp_idx, conv1_w, conv1_b, conv2_w, conv2_b,
             enc_wih, enc_whh, enc_bih, enc_bhh,
             dec_wih, dec_whh, dec_bih, dec_bhh,
             px_w1, px_b1, px_w2, px_b2,
             py_w1, py_b1, py_w2, py_b2, *, interpret=False):
    B, C, H, W = x.shape
    N, K = kp_idx.shape[1], kp_idx.shape[2]
    D = conv2_w.shape[0]
    HID = enc_whh.shape[1]

    x_t = jnp.transpose(x, (0, 2, 3, 1))
    idx = jnp.clip(kp_idx, 0, W - 1).astype(jnp.int32)
    i0 = idx[..., 0].reshape(B, N * K)
    i1 = idx[..., 1].reshape(B, N * K)
    lin = jnp.concatenate([i0 * W + i1, i1 * W + i0], axis=1)
    idx_flat = idx.reshape(B, N * K * 2)
    w1r = conv1_w.transpose(2, 3, 1, 0).reshape(9, C, C)

    g_all = _conv_gather(x_t, lin, w1r, conv1_b.reshape(1, C), conv2_w.T,
                         interpret=interpret)

    seq = g_all.reshape(B, 2, N, K, D).transpose(3, 1, 0, 2, 4)
    seq = seq.reshape(K, 2 * B * N, D)
    pw2p = jnp.zeros((py_w1.shape[0], 128), jnp.float32).at[:, 0].set(py_w2[0])
    y, gt = _lstm_mlp(seq, idx_flat, conv2_b.reshape(1, D),
                      enc_wih.T, enc_whh.T,
                      (enc_bih + enc_bhh).reshape(1, -1),
                      dec_wih.T, dec_whh.T, (dec_bih + dec_bhh).reshape(1, -1),
                      py_w1.T, py_b1.reshape(1, -1), pw2p, py_b2,
                      interpret=interpret)

    keypoints = y[:, 0].reshape(K, 2, B, N).transpose(2, 3, 0, 1)
    gt_keypoints = gt.reshape(B, N, K, 2)
    return keypoints, gt_keypoints


def kernel(x, kp_idx, conv1_w, conv1_b, conv2_w, conv2_b,
           enc_wih, enc_whh, enc_bih, enc_bhh,
           dec_wih, dec_whh, dec_bih, dec_bhh,
           px_w1, px_b1, px_w2, px_b2,
           py_w1, py_b1, py_w2, py_b2):
    return _forward(x, kp_idx, conv1_w, conv1_b, conv2_w, conv2_b,
                    enc_wih, enc_whh, enc_bih, enc_bhh,
                    dec_wih, dec_whh, dec_bih, dec_bhh,
                    px_w1, px_b1, px_w2, px_b2,
                    py_w1, py_b1, py_w2, py_b2)


# speedup vs baseline: 1.0943x; 1.0943x over previous
import jax
import jax.numpy as jnp
from jax.experimental import pallas as pl
from jax.experimental.pallas import tpu as pltpu


def _compiler_params(dims):
    return pltpu.CompilerParams(
        dimension_semantics=dims,
        vmem_limit_bytes=60 * 1024 * 1024,
    )


def _conv_gather_kernel(lin_ref, x_ref, w1_ref, b1_ref, w2_ref,
                        g_ref, feat_ref):
    b = pl.program_id(0)
    H = x_ref.shape[1] - 8
    W = x_ref.shape[2] - 8
    C = x_ref.shape[3]
    D = w2_ref.shape[1]
    M = g_ref.shape[1]

    acc = None
    for dy in range(3):
        for dx in range(3):
            xs = x_ref[0, dy:dy + H, dx:dx + W, :].reshape(H * W, C)
            d = jnp.dot(xs, w1_ref[dy * 3 + dx],
                        preferred_element_type=jnp.float32)
            acc = d if acc is None else acc + d
    f1 = jnp.maximum(acc + b1_ref[...], 0.0)
    feat_ref[...] = jnp.dot(f1, w2_ref[...],
                            preferred_element_type=jnp.float32)

    iota8 = jax.lax.broadcasted_iota(jnp.int32, (8, D), 0)

    def outer(o, carry):
        m0 = pl.multiple_of(o * 16, 8)
        rows = []
        for i in range(16):
            r = lin_ref[b, m0 + i]
            base = pl.multiple_of((r >> 3) << 3, 8)
            chunk = feat_ref[pl.ds(base, 8), :]
            msk = (iota8 == (r & 7)).astype(jnp.float32)
            rows.append(jnp.sum(chunk * msk, axis=0, keepdims=True))
        g_ref[0, pl.ds(m0, 16), :] = jnp.concatenate(rows, axis=0)
        return carry

    jax.lax.fori_loop(0, M // 16, outer, 0)


def _conv_gather(x_pad, lin, w1r, b1r, w2t, *, interpret=False):
    B, Hp, Wp, C = x_pad.shape
    H, W = Hp - 8, Wp - 8
    M = lin.shape[1]
    D = w2t.shape[1]
    f32 = jnp.float32
    return pl.pallas_call(
        _conv_gather_kernel,
        grid=(B,),
        in_specs=[
            pl.BlockSpec(memory_space=pltpu.SMEM),
            pl.BlockSpec((1, Hp, Wp, C), lambda b: (b, 0, 0, 0)),
            pl.BlockSpec((9, C, C), lambda b: (0, 0, 0)),
            pl.BlockSpec((1, C), lambda b: (0, 0)),
            pl.BlockSpec((C, D), lambda b: (0, 0)),
        ],
        out_specs=pl.BlockSpec((1, M, D), lambda b: (b, 0, 0)),
        out_shape=jax.ShapeDtypeStruct((B, M, D), f32),
        scratch_shapes=[pltpu.VMEM((H * W, D), f32)],
        compiler_params=_compiler_params(("parallel",)),
        interpret=interpret,
    )(lin, x_pad, w1r, b1r, w2t)


def _lstm_kernel(seq_ref, idx_ref, b2_ref, ewih_ref, ewhh_ref, eb_ref,
                 dwih_ref, dwhh_ref, db_ref, pw1_ref, pb1_ref, pw2_ref,
                 pb2_ref, y_ref, gt_ref, enc_ref):
    T = seq_ref.shape[0]
    Sh = seq_ref.shape[1]
    HID = ewhh_ref.shape[0]

    gt_ref[...] = idx_ref[...].astype(jnp.float32) * 4.0

    def sig(v):
        return jnp.tanh(v * 0.5) * 0.5 + 0.5

    def gates(xt, h, c, wih, whh, bias):
        g = (jnp.dot(xt, wih, preferred_element_type=jnp.float32)
             + jnp.dot(h, whh, preferred_element_type=jnp.float32)
             + bias)
        ig = sig(g[:, 0:HID])
        fg = sig(g[:, HID:2 * HID])
        gg = jnp.tanh(g[:, 2 * HID:3 * HID])
        og = sig(g[:, 3 * HID:4 * HID])
        c = fg * c + ig * gg
        h = og * jnp.tanh(c)
        return h, c

    def enc_step(t, hc):
        h, c = gates(seq_ref[t] + b2_ref[...], hc[0], hc[1], ewih_ref[...],
                     ewhh_ref[...], eb_ref[...])
        enc_ref[t] = h
        return (h, c)

    z = jnp.zeros((Sh, HID), jnp.float32)
    hc = jax.lax.fori_loop(0, T, enc_step, (z, z))

    def dec_step(t, hc):
        h, c = gates(enc_ref[t], hc[0], hc[1], dwih_ref[...], dwhh_ref[...],
                     db_ref[...])
        m = jnp.maximum(jnp.dot(h, pw1_ref[...],
                                preferred_element_type=jnp.float32)
                        + pb1_ref[...], 0.0)
        y = jnp.dot(m, pw2_ref[...],
                    preferred_element_type=jnp.float32) + pb2_ref[0]
        t0 = pl.multiple_of(t * Sh, 8)
        y_ref[pl.ds(t0, Sh), :] = y
        return (h, c)

    jax.lax.fori_loop(0, T, dec_step, hc)


def _lstm_mlp(seq, idx_flat, b2r, ewih_t, ewhh_t, eb, dwih_t, dwhh_t, db,
              pw1t, pb1, pw2p, pb2, *, interpret=False):
    T, S, D = seq.shape
    HID = ewhh_t.shape[0]
    f32 = jnp.float32
    vmem = [pl.BlockSpec(memory_space=pltpu.VMEM)] * 12
    return pl.pallas_call(
        _lstm_kernel,
        in_specs=vmem + [pl.BlockSpec(memory_space=pltpu.SMEM)],
        out_specs=[pl.BlockSpec(memory_space=pltpu.VMEM),
                   pl.BlockSpec(memory_space=pltpu.VMEM)],
        out_shape=[jax.ShapeDtypeStruct((T * S, 128), f32),
                   jax.ShapeDtypeStruct(idx_flat.shape, f32)],
        scratch_shapes=[pltpu.VMEM((T, S, HID), f32)],
        compiler_params=_compiler_params(()),
        interpret=interpret,
    )(seq, idx_flat, b2r, ewih_t, ewhh_t, eb, dwih_t, dwhh_t, db, pw1t, pb1,
      pw2p, pb2)


def _forward(x, kp_idx, conv1_w, conv1_b, conv2_w, conv2_b,
             enc_wih, enc_whh, enc_bih, enc_bhh,
             dec_wih, dec_whh, dec_bih, dec_bhh,
             px_w1, px_b1, px_w2, px_b2,
             py_w1, py_b1, py_w2, py_b2, *, interpret=False):
    B, C, H, W = x.shape
    N, K = kp_idx.shape[1], kp_idx.shape[2]
    D = conv2_w.shape[0]
    HID = enc_whh.shape[1]

    x_pad = jnp.pad(jnp.transpose(x, (0, 2, 3, 1)),
                    ((0, 0), (1, 7), (1, 7), (0, 0)))
    idx = jnp.clip(kp_idx, 0, W - 1).astype(jnp.int32)
    i0 = idx[..., 0].reshape(B, N * K)
    i1 = idx[..., 1].reshape(B, N * K)
    lin = jnp.concatenate([i0 * W + i1, i1 * W + i0], axis=1)
    idx_flat = idx.reshape(B, N * K * 2)
    w1r = conv1_w.transpose(2, 3, 1, 0).reshape(9, C, C)

    g_all = _conv_gather(x_pad, lin, w1r, conv1_b.reshape(1, C), conv2_w.T,
                         interpret=interpret)

    seq = g_all.reshape(B, 2, N, K, D).transpose(3, 1, 0, 2, 4)
    seq = seq.reshape(K, 2 * B * N, D)
    pw2p = jnp.zeros((py_w1.shape[0], 128), jnp.float32).at[:, 0].set(py_w2[0])
    y, gt = _lstm_mlp(seq, idx_flat, conv2_b.reshape(1, D),
                      enc_wih.T, enc_whh.T,
                      (enc_bih + enc_bhh).reshape(1, -1),
                      dec_wih.T, dec_whh.T, (dec_bih + dec_bhh).reshape(1, -1),
                      py_w1.T, py_b1.reshape(1, -1), pw2p, py_b2,
                      interpret=interpret)

    keypoints = y[:, 0].reshape(K, 2, B, N).transpose(2, 3, 0, 1)
    gt_keypoints = gt.reshape(B, N, K, 2)
    return keypoints, gt_keypoints


def kernel(x, kp_idx, conv1_w, conv1_b, conv2_w, conv2_b,
           enc_wih, enc_whh, enc_bih, enc_bhh,
           dec_wih, dec_whh, dec_bih, dec_bhh,
           px_w1, px_b1, px_w2, px_b2,
           py_w1, py_b1, py_w2, py_b2):
    return _forward(x, kp_idx, conv1_w, conv1_b, conv2_w, conv2_b,
                    enc_wih, enc_whh, enc_bih, enc_bhh,
                    dec_wih, dec_whh, dec_bih, dec_bhh,
                    px_w1, px_b1, px_w2, px_b2,
                    py_w1, py_b1, py_w2, py_b2)
